# baseline (speedup 1.0000x reference)
import jax
import jax.numpy as jnp
from jax.experimental import pallas as pl
from jax.experimental.pallas import tpu as pltpu

_NT = 256


def _chamfer_tc_kernel(x_ref, yt_ref, row_tot_ref, col_tot_ref, colmin_ref):
    b = pl.program_id(0)
    i = pl.program_id(1)
    ni = pl.num_programs(1)

    x = x_ref[0]
    yt = yt_ref[0]

    x0, x1, x2c = x[:, 0:1], x[:, 1:2], x[:, 2:3]
    y0, y1, y2c = yt[0:1, :], yt[1:2, :], yt[2:3, :]
    xsq = x0 * x0 + x1 * x1 + x2c * x2c
    ysq = y0 * y0 + y1 * y1 + y2c * y2c

    def _rt(v):
        return v.astype(jnp.bfloat16).astype(jnp.float32)

    inner = _rt(x0) * _rt(y0) + _rt(x1) * _rt(y1) + _rt(x2c) * _rt(y2c)
    d = xsq + ysq - 2.0 * inner

    row_min = jnp.min(d, axis=1, keepdims=True)
    col_min = jnp.min(d, axis=0, keepdims=True)

    @pl.when(jnp.logical_and(b == 0, i == 0))
    def _init():
        row_tot_ref[...] = jnp.zeros((1, 1), jnp.float32)
        col_tot_ref[...] = jnp.zeros((1, 1), jnp.float32)

    row_tot_ref[...] += jnp.sum(row_min, axis=0, keepdims=True)

    @pl.when(i == 0)
    def _colmin_init():
        colmin_ref[...] = col_min

    @pl.when(i > 0)
    def _colmin_acc():
        colmin_ref[...] = jnp.minimum(colmin_ref[...], col_min)

    @pl.when(i == ni - 1)
    def _colmin_finish():
        col_tot_ref[...] += jnp.sum(colmin_ref[...], axis=1, keepdims=True)


def kernel(pred, gt):
    B, N, D = pred.shape
    M = gt.shape[1]
    gt_t = jnp.swapaxes(gt, 1, 2)

    grid = (B, N // _NT)
    row_tot, col_tot = pl.pallas_call(
        _chamfer_tc_kernel,
        grid=grid,
        in_specs=[
            pl.BlockSpec((1, _NT, D), lambda b, i: (b, i, 0)),
            pl.BlockSpec((1, D, M), lambda b, i: (b, 0, 0)),
        ],
        out_specs=[
            pl.BlockSpec((1, 1), lambda b, i: (0, 0)),
            pl.BlockSpec((1, 1), lambda b, i: (0, 0)),
        ],
        out_shape=[
            jax.ShapeDtypeStruct((1, 1), jnp.float32),
            jax.ShapeDtypeStruct((1, 1), jnp.float32),
        ],
        scratch_shapes=[pltpu.VMEM((1, M), jnp.float32)],
    )(pred, gt_t)

    return row_tot[0, 0] / (B * N) + col_tot[0, 0] / (B * M)


# speedup vs baseline: 1.2373x; 1.2373x over previous
import jax
import jax.numpy as jnp
from jax.experimental import pallas as pl
from jax.experimental.pallas import tpu as pltpu

_NT = 512
_K = 8


def _chamfer_tc_kernel(xa_ref, ya_ref, row_tot_ref, col_tot_ref, colmin_ref):
    b = pl.program_id(0)
    i = pl.program_id(1)
    ni = pl.num_programs(1)

    d = jax.lax.dot_general(
        xa_ref[0], ya_ref[0],
        dimension_numbers=(((1,), (0,)), ((), ())),
        preferred_element_type=jnp.float32,
    )

    row_min = jnp.min(d, axis=1, keepdims=True)
    col_min = jnp.min(d, axis=0, keepdims=True)

    @pl.when(jnp.logical_and(b == 0, i == 0))
    def _init():
        row_tot_ref[...] = jnp.zeros((1, 1), jnp.float32)
        col_tot_ref[...] = jnp.zeros((1, 1), jnp.float32)

    row_tot_ref[...] += jnp.sum(row_min, axis=0, keepdims=True)

    @pl.when(i == 0)
    def _colmin_init():
        colmin_ref[...] = col_min

    @pl.when(i > 0)
    def _colmin_acc():
        colmin_ref[...] = jnp.minimum(colmin_ref[...], col_min)

    @pl.when(i == ni - 1)
    def _colmin_finish():
        col_tot_ref[...] += jnp.sum(colmin_ref[...], axis=1, keepdims=True)


def _split_hi_lo(v):
    hi = v.astype(jnp.bfloat16)
    lo = (v - hi.astype(jnp.float32)).astype(jnp.bfloat16)
    return hi, lo


def kernel(pred, gt):
    B, N, D = pred.shape
    M = gt.shape[1]
    f32 = jnp.float32
    bf16 = jnp.bfloat16

    xsq = jnp.sum(pred * pred, axis=-1)
    ysq = jnp.sum(gt * gt, axis=-1)
    xsq_hi, xsq_lo = _split_hi_lo(xsq)
    ysq_hi, ysq_lo = _split_hi_lo(ysq)
    ones_x = jnp.ones((B, N), bf16)
    ones_y = jnp.ones((B, M), bf16)
    zeros_x = jnp.zeros((B, N), bf16)
    zeros_y = jnp.zeros((B, M), bf16)

    xa = jnp.stack(
        [(-2.0 * pred[..., k].astype(bf16).astype(f32)).astype(bf16)
         for k in range(D)]
        + [xsq_hi, xsq_lo, ones_x, ones_x, zeros_x],
        axis=-1,
    )
    ya = jnp.stack(
        [gt[..., k].astype(bf16) for k in range(D)]
        + [ones_y, ones_y, ysq_hi, ysq_lo, zeros_y],
        axis=1,
    )

    grid = (B, N // _NT)
    row_tot, col_tot = pl.pallas_call(
        _chamfer_tc_kernel,
        grid=grid,
        in_specs=[
            pl.BlockSpec((1, _NT, _K), lambda b, i: (b, i, 0)),
            pl.BlockSpec((1, _K, M), lambda b, i: (b, 0, 0)),
        ],
        out_specs=[
            pl.BlockSpec((1, 1), lambda b, i: (0, 0)),
            pl.BlockSpec((1, 1), lambda b, i: (0, 0)),
        ],
        out_shape=[
            jax.ShapeDtypeStruct((1, 1), jnp.float32),
            jax.ShapeDtypeStruct((1, 1), jnp.float32),
        ],
        scratch_shapes=[pltpu.VMEM((1, M), jnp.float32)],
    )(xa, ya)

    return row_tot[0, 0] / (B * N) + col_tot[0, 0] / (B * M)


# speedup vs baseline: 1.2878x; 1.0408x over previous
import jax
import jax.numpy as jnp
from jax.experimental import pallas as pl
from jax.experimental.pallas import tpu as pltpu

_NT = 1024
_K = 8


def _chamfer_tc_kernel(xa_ref, ya_ref, row_tot_ref, col_tot_ref, colmin_ref):
    b = pl.program_id(0)
    i = pl.program_id(1)
    ni = pl.num_programs(1)

    d = jax.lax.dot_general(
        xa_ref[0], ya_ref[0],
        dimension_numbers=(((1,), (0,)), ((), ())),
        preferred_element_type=jnp.float32,
    )

    row_min = jnp.min(d, axis=1, keepdims=True)
    col_min = jnp.min(d, axis=0, keepdims=True)

    @pl.when(jnp.logical_and(b == 0, i == 0))
    def _init():
        row_tot_ref[...] = jnp.zeros((1, 1), jnp.float32)
        col_tot_ref[...] = jnp.zeros((1, 1), jnp.float32)

    row_tot_ref[...] += jnp.sum(row_min, axis=0, keepdims=True)

    @pl.when(i == 0)
    def _colmin_init():
        colmin_ref[...] = col_min

    @pl.when(i > 0)
    def _colmin_acc():
        colmin_ref[...] = jnp.minimum(colmin_ref[...], col_min)

    @pl.when(i == ni - 1)
    def _colmin_finish():
        col_tot_ref[...] += jnp.sum(colmin_ref[...], axis=1, keepdims=True)


def _split_hi_lo(v):
    hi = v.astype(jnp.bfloat16)
    lo = (v - hi.astype(jnp.float32)).astype(jnp.bfloat16)
    return hi, lo


def kernel(pred, gt):
    B, N, D = pred.shape
    M = gt.shape[1]
    f32 = jnp.float32
    bf16 = jnp.bfloat16

    xsq = jnp.sum(pred * pred, axis=-1)
    ysq = jnp.sum(gt * gt, axis=-1)
    xsq_hi, xsq_lo = _split_hi_lo(xsq)
    ysq_hi, ysq_lo = _split_hi_lo(ysq)
    ones_x = jnp.ones((B, N), bf16)
    ones_y = jnp.ones((B, M), bf16)
    zeros_x = jnp.zeros((B, N), bf16)
    zeros_y = jnp.zeros((B, M), bf16)

    xa = jnp.stack(
        [(-2.0 * pred[..., k].astype(bf16).astype(f32)).astype(bf16)
         for k in range(D)]
        + [xsq_hi, xsq_lo, ones_x, ones_x, zeros_x],
        axis=-1,
    )
    ya = jnp.stack(
        [gt[..., k].astype(bf16) for k in range(D)]
        + [ones_y, ones_y, ysq_hi, ysq_lo, zeros_y],
        axis=1,
    )

    grid = (B, N // _NT)
    row_tot, col_tot = pl.pallas_call(
        _chamfer_tc_kernel,
        grid=grid,
        in_specs=[
            pl.BlockSpec((1, _NT, _K), lambda b, i: (b, i, 0)),
            pl.BlockSpec((1, _K, M), lambda b, i: (b, 0, 0)),
        ],
        out_specs=[
            pl.BlockSpec((1, 1), lambda b, i: (0, 0)),
            pl.BlockSpec((1, 1), lambda b, i: (0, 0)),
        ],
        out_shape=[
            jax.ShapeDtypeStruct((1, 1), jnp.float32),
            jax.ShapeDtypeStruct((1, 1), jnp.float32),
        ],
        scratch_shapes=[pltpu.VMEM((1, M), jnp.float32)],
    )(xa, ya)

    return row_tot[0, 0] / (B * N) + col_tot[0, 0] / (B * M)


# speedup vs baseline: 2.4730x; 1.9204x over previous
import jax
import jax.numpy as jnp
from jax.experimental import pallas as pl
from jax.experimental.pallas import tpu as pltpu

_NT = 1024


def _chamfer_tc_kernel(x_ref, yt_ref, row_tot_ref, col_tot_ref, colmin_ref):
    b = pl.program_id(0)
    i = pl.program_id(1)
    ni = pl.num_programs(1)

    x = x_ref[0]
    yt = yt_ref[0]

    xsq = jnp.sum(x * x, axis=1, keepdims=True)
    ysq = jnp.sum(yt * yt, axis=0, keepdims=True)

    xm = (-2.0 * x).astype(jnp.bfloat16)
    ytb = yt.astype(jnp.bfloat16)
    inner_m2 = jax.lax.dot_general(
        xm, ytb,
        dimension_numbers=(((1,), (0,)), ((), ())),
        preferred_element_type=jnp.float32,
    )

    d = (xsq + ysq) + inner_m2

    row_min = jnp.min(d, axis=1, keepdims=True)
    col_min = jnp.min(d, axis=0, keepdims=True)

    @pl.when(jnp.logical_and(b == 0, i == 0))
    def _init():
        row_tot_ref[...] = jnp.zeros((1, 1), jnp.float32)
        col_tot_ref[...] = jnp.zeros((1, 1), jnp.float32)

    row_tot_ref[...] += jnp.sum(row_min, axis=0, keepdims=True)

    @pl.when(i == 0)
    def _colmin_init():
        colmin_ref[...] = col_min

    @pl.when(i > 0)
    def _colmin_acc():
        colmin_ref[...] = jnp.minimum(colmin_ref[...], col_min)

    @pl.when(i == ni - 1)
    def _colmin_finish():
        col_tot_ref[...] += jnp.sum(colmin_ref[...], axis=1, keepdims=True)


def kernel(pred, gt):
    B, N, D = pred.shape
    M = gt.shape[1]
    gt_t = jnp.swapaxes(gt, 1, 2)

    grid = (B, N // _NT)
    row_tot, col_tot = pl.pallas_call(
        _chamfer_tc_kernel,
        grid=grid,
        in_specs=[
            pl.BlockSpec((1, _NT, D), lambda b, i: (b, i, 0)),
            pl.BlockSpec((1, D, M), lambda b, i: (b, 0, 0)),
        ],
        out_specs=[
            pl.BlockSpec((1, 1), lambda b, i: (0, 0)),
            pl.BlockSpec((1, 1), lambda b, i: (0, 0)),
        ],
        out_shape=[
            jax.ShapeDtypeStruct((1, 1), jnp.float32),
            jax.ShapeDtypeStruct((1, 1), jnp.float32),
        ],
        scratch_shapes=[pltpu.VMEM((1, M), jnp.float32)],
    )(pred, gt_t)

    return row_tot[0, 0] / (B * N) + col_tot[0, 0] / (B * M)
